# baseline (speedup 1.0000x reference)
import functools

import jax
import jax.numpy as jnp
from jax import lax
from jax.experimental import pallas as pl
from jax.experimental.pallas import tpu as pltpu
from jax.experimental.pallas import tpu_sc as plsc

RES = 256
NF = 1048576
FD = 16
RAD = 1.0
NI = 32
SS = 0.0625

B = 4096
P = B * NI
N1 = P * 8
N2 = P * 16
NC, NS = 2, 16
NW = NC * NS
T128 = 128

N1R = N1 // T128
N2R = N2 // T128
W1R = N1R // NW
W2R = N2R // NW

_HI = jax.lax.Precision.HIGHEST


def _sc_gather_scalar(gtab, idx):
    mesh = plsc.VectorSubcoreMesh(core_axis_name="c", subcore_axis_name="s")

    @functools.partial(
        pl.kernel, mesh=mesh,
        out_type=jax.ShapeDtypeStruct((N1R, T128), jnp.float32),
        compiler_params=pltpu.CompilerParams(use_tc_tiling_on_sc=False),
        scratch_types=[
            pltpu.VMEM((W1R, T128), jnp.int32),
            pltpu.VMEM((W1R, T128), jnp.float32),
            pltpu.SemaphoreType.DMA,
        ],
    )
    def k(gt_hbm, idx_hbm, out_hbm, idx_v, out_v, sem):
        wid = lax.axis_index("s") * NC + lax.axis_index("c")
        base = wid * W1R
        pltpu.sync_copy(idx_hbm.at[pl.ds(base, W1R)], idx_v)

        def macro(m, carry):
            cps = []
            for u in range(16):
                j = m * 16 + u
                cps.append(pltpu.async_copy(
                    gt_hbm.at[idx_v.at[j]], out_v.at[j], sem))
            for c in cps:
                c.wait()
            return carry

        lax.fori_loop(0, W1R // 16, macro, 0)
        pltpu.sync_copy(out_v, out_hbm.at[pl.ds(base, W1R)])

    return k(gtab, idx)


def _sc_gather_reduce(ftab, idx, wcf):
    mesh = plsc.VectorSubcoreMesh(core_axis_name="c", subcore_axis_name="s")

    @functools.partial(
        pl.kernel, mesh=mesh,
        out_type=jax.ShapeDtypeStruct((P, T128), jnp.float32),
        compiler_params=pltpu.CompilerParams(use_tc_tiling_on_sc=False),
        scratch_types=[
            pltpu.VMEM((W2R, T128), jnp.int32),
            pltpu.VMEM((16 * T128, FD), jnp.float32),
            pltpu.VMEM((16, T128), jnp.float32),
            pltpu.VMEM((T128, FD), jnp.float32),
            pltpu.SemaphoreType.DMA,
        ],
    )
    def k(ft_hbm, idx_hbm, wc_hbm, out_hbm, idx_v, rows_v, wc_v, fv_v, sem):
        wid = lax.axis_index("s") * NC + lax.axis_index("c")
        ibase = wid * W2R
        pltpu.sync_copy(idx_hbm.at[pl.ds(ibase, W2R)], idx_v)

        def macro(m, carry):
            cps = []
            for u in range(16):
                cps.append(pltpu.async_copy(
                    ft_hbm.at[idx_v.at[m * 16 + u]],
                    rows_v.at[pl.ds(u * T128, T128)], sem))
            pltpu.sync_copy(wc_hbm.at[pl.ds(ibase + m * 16, 16)], wc_v)
            for c in cps:
                c.wait()

            def point(p, carry2):
                wvec = wc_v[p // 8, pl.ds((p % 8) * 16, 16)]
                base = p * 16
                acc = wvec[0] * rows_v[base, :]
                for j in range(1, 16):
                    acc = acc + wvec[j] * rows_v[base + j, :]
                fv_v[p, :] = acc
                return carry2

            lax.fori_loop(0, T128, point, 0)
            pltpu.sync_copy(
                fv_v,
                out_hbm.at[pl.ds(wid * (P // NW) + m * T128, T128),
                           pl.ds(0, FD)])
            return carry

        lax.fori_loop(0, W2R // 16, macro, 0)

    return k(ftab, idx, wcf)


_TC1_BLOCKS = 64
_PB = P // _TC1_BLOCKS
_RB = _PB * 16


def _tc1_body(fv_ref, enc_ref, sw1_ref, sw2_ref, cw1a_ref,
              cw1b_ref, cw2_ref, cw3_ref, fo_ref, col_ref):
    fvals = fv_ref[:, 0:FD]
    encr = enc_ref[...]
    enc = jnp.broadcast_to(encr[:, None, :],
                           (_PB // NI, NI, 16)).reshape(_PB, 16)
    h = jax.nn.relu(jnp.dot(fvals, sw1_ref[...], precision=_HI))
    fo = jnp.dot(h, sw2_ref[...], precision=_HI)
    c_pre = (jnp.dot(enc, cw1a_ref[...], precision=_HI)
             + jnp.dot(fo, cw1b_ref[...], precision=_HI))
    hc = jax.nn.relu(c_pre)
    hc2 = jax.nn.relu(jnp.dot(hc, cw2_ref[...], precision=_HI))
    col = jnp.dot(hc2, cw3_ref[...], precision=_HI)
    fo_ref[...] = fo
    col_ref[...] = col


def _tc1(fv, encr, sw1, sw2, cw1a, cw1b, cw2, cw3):
    rep2 = lambda shape: pl.BlockSpec(shape, lambda i: (0, 0))
    return pl.pallas_call(
        _tc1_body,
        grid=(_TC1_BLOCKS,),
        in_specs=[
            pl.BlockSpec((_PB, T128), lambda i: (i, 0)),
            pl.BlockSpec((_PB // NI, 16), lambda i: (i, 0)),
            rep2((16, 64)), rep2((64, 16)), rep2((16, 64)),
            rep2((16, 64)), rep2((64, 64)), rep2((64, 3)),
        ],
        out_specs=[
            pl.BlockSpec((_PB, 16), lambda i: (i, 0)),
            pl.BlockSpec((_PB, 3), lambda i: (i, 0)),
        ],
        out_shape=[
            jax.ShapeDtypeStruct((P, 16), jnp.float32),
            jax.ShapeDtypeStruct((P, 3), jnp.float32),
        ],
    )(fv, encr, sw1, sw2, cw1a, cw1b, cw2, cw3)


_TC2_BLOCKS = 32
_RAYB = B // _TC2_BLOCKS


def _tc2_body(sig_ref, mask_ref, c0_ref, c1_ref, c2_ref, out_ref):
    maskv = mask_ref[...]
    sigma = jnp.where(maskv > 0, sig_ref[...], 0.0)
    alpha = 1.0 - jnp.exp(-jax.nn.relu(sigma) * SS)
    x = 1.0 - alpha + 1e-10
    lx = jnp.log(x)
    r = lax.broadcasted_iota(jnp.int32, (NI, NI), 0)
    c = lax.broadcasted_iota(jnp.int32, (NI, NI), 1)
    m_excl = (r < c).astype(jnp.float32)
    t_tr = jnp.exp(jnp.dot(lx, m_excl, precision=_HI))
    al = alpha * t_tr
    bg = 1.0 - jnp.sum(al, axis=1, keepdims=True)
    outs = []
    for cref in (c0_ref, c1_ref, c2_ref):
        s = jax.nn.sigmoid(cref[...] * maskv)
        outs.append(jnp.sum(al * s, axis=1, keepdims=True) + bg)
    out_ref[...] = jnp.concatenate(outs, axis=1)


def _tc2(sig, maskf, c0, c1, c2):
    spec = pl.BlockSpec((_RAYB, NI), lambda i: (i, 0))
    return pl.pallas_call(
        _tc2_body,
        grid=(_TC2_BLOCKS,),
        in_specs=[spec, spec, spec, spec, spec],
        out_specs=pl.BlockSpec((_RAYB, 3), lambda i: (i, 0)),
        out_shape=jax.ShapeDtypeStruct((B, 3), jnp.float32),
    )(sig, maskf, c0, c1, c2)


def _tri_weights(o):
    x, y, z = o[:, 0], o[:, 1], o[:, 2]
    return jnp.stack([
        (1 - x) * (1 - y) * (1 - z),
        (1 - x) * (1 - y) * z,
        (1 - x) * y * (1 - z),
        (1 - x) * y * z,
        x * (1 - y) * (1 - z),
        x * (1 - y) * z,
        x * y * (1 - z),
        x * y * z,
    ], axis=-1)


def _sh16_enc(d):
    x, y, z = d[:, 0], d[:, 1], d[:, 2]
    xx, yy, zz = x * x, y * y, z * z
    return jnp.stack([
        0.28209479177387814 * jnp.ones_like(x),
        -0.48860251190291987 * y,
        0.48860251190291987 * z,
        -0.48860251190291987 * x,
        1.0925484305920792 * x * y,
        -1.0925484305920792 * y * z,
        0.31539156525252005 * (3.0 * zz - 1.0),
        -1.0925484305920792 * x * z,
        0.5462742152960396 * (xx - yy),
        -0.5900435899266435 * y * (3.0 * xx - yy),
        2.890611442640554 * x * y * z,
        -0.4570457994644658 * y * (4.0 * zz - xx - yy),
        0.3731763325901154 * z * (2.0 * zz - 3.0 * xx - 3.0 * yy),
        -0.4570457994644658 * x * (4.0 * zz - xx - yy),
        1.445305721320277 * z * (xx - yy),
        -0.5900435899266435 * x * (xx - 3.0 * yy),
    ], axis=-1)


def kernel(rays_o, rays_d, G, F, sigma_W1, sigma_W2, color_W1, color_W2,
           color_W3):
    rd = rays_d / jnp.linalg.norm(rays_d, axis=-1, keepdims=True)
    t = jnp.arange(NI + 1, dtype=jnp.float32) * SS
    tm = 0.5 * (t[:-1] + t[1:])
    pts = rays_o[:, None, :] + rd[:, None, :] * tm[None, :, None]
    mask = jnp.linalg.norm(pts, axis=-1) < RAD
    fp = ((pts / RAD + 1.0) * RES / 2.0).reshape(-1, 3)

    offs3 = jnp.array([[-1, -1, -1], [-1, -1, 1], [-1, 1, -1], [-1, 1, 1],
                       [1, -1, -1], [1, -1, 1], [1, 1, -1], [1, 1, 1]],
                      dtype=jnp.float32)
    pre = fp[:, None, :] + offs3[None, ...] / 2.0
    pf = jnp.clip(jnp.floor(pre), 0.0, RES - 1)
    offsets = fp - pf[:, 0, :]
    nb = pf.astype(jnp.int32)
    gidx = (nb[:, :, 0] * (RES * RES) + nb[:, :, 1] * RES + nb[:, :, 2])
    w8 = _tri_weights(offsets)

    gv = _sc_gather_scalar(G.reshape(-1), gidx.reshape(N1R, T128))
    gv = gv.reshape(P, 8)

    fi = (jnp.clip(gv, -1.0, 1.0) + 1.0) * (NF / 2.0)
    fl = jnp.clip(jnp.floor(fi), 0.0, NF - 1)
    ce = jnp.clip(jnp.ceil(fi), 0.0, NF - 1)
    wlo = w8 * (ce - fi)
    whi = w8 * (fi - fl)
    wcf = jnp.stack([wlo, whi], axis=-1).reshape(N2R, T128)
    idx2 = jnp.stack([fl.astype(jnp.int32), ce.astype(jnp.int32)],
                     axis=-1).reshape(N2R, T128)

    fv = _sc_gather_reduce(F, idx2, wcf)

    enc = _sh16_enc(rd)
    cw1b = jnp.concatenate([jnp.zeros((1, 64), jnp.float32), color_W1[16:]],
                           axis=0)
    fo, col = _tc1(fv, enc, sigma_W1, sigma_W2, color_W1[:16],
                   cw1b, color_W2, color_W3)

    maskf = mask.astype(jnp.float32)
    sig = fo[:, 0].reshape(B, NI)
    c0 = col[:, 0].reshape(B, NI)
    c1 = col[:, 1].reshape(B, NI)
    c2 = col[:, 2].reshape(B, NI)
    return _tc2(sig, maskf, c0, c1, c2)


# speedup vs baseline: 2.2451x; 2.2451x over previous
import functools

import jax
import jax.numpy as jnp
from jax import lax
from jax.experimental import pallas as pl
from jax.experimental.pallas import tpu as pltpu
from jax.experimental.pallas import tpu_sc as plsc

RES = 256
NF = 1048576
FD = 16
RAD = 1.0
NI = 32
SS = 0.0625

B = 4096
P = B * NI
N1 = P * 8
N2 = P * 16
NC, NS = 2, 16
NW = NC * NS
T128 = 128

N1R = N1 // T128
N2R = N2 // T128
W1R = N1R // NW
W2R = N2R // NW

_HI = jax.lax.Precision.HIGHEST


def _sc_gather_scalar(gtab, idx):
    mesh = plsc.VectorSubcoreMesh(core_axis_name="c", subcore_axis_name="s")

    @functools.partial(
        pl.kernel, mesh=mesh,
        out_type=jax.ShapeDtypeStruct((N1R, T128), jnp.float32),
        compiler_params=pltpu.CompilerParams(use_tc_tiling_on_sc=False),
        scratch_types=[
            pltpu.VMEM((W1R, T128), jnp.int32),
            pltpu.VMEM((W1R, T128), jnp.float32),
            pltpu.SemaphoreType.DMA,
        ],
    )
    def k(gt_hbm, idx_hbm, out_hbm, idx_v, out_v, sem):
        wid = lax.axis_index("s") * NC + lax.axis_index("c")
        base = wid * W1R
        pltpu.sync_copy(idx_hbm.at[pl.ds(base, W1R)], idx_v)

        def macro(m, carry):
            cps = []
            for u in range(16):
                j = m * 16 + u
                cps.append(pltpu.async_copy(
                    gt_hbm.at[idx_v.at[j]], out_v.at[j], sem))
            for c in cps:
                c.wait()
            return carry

        lax.fori_loop(0, W1R // 16, macro, 0)
        pltpu.sync_copy(out_v, out_hbm.at[pl.ds(base, W1R)])

    return k(gtab, idx)


def _sc_gather_reduce(ftab, idx, wcf):
    mesh = plsc.VectorSubcoreMesh(core_axis_name="c", subcore_axis_name="s")

    @functools.partial(
        pl.kernel, mesh=mesh,
        out_type=jax.ShapeDtypeStruct((P, T128), jnp.float32),
        compiler_params=pltpu.CompilerParams(use_tc_tiling_on_sc=False),
        scratch_types=[
            pltpu.VMEM((W2R, T128), jnp.int32),
            pltpu.VMEM((16 * T128, FD), jnp.float32),
            pltpu.VMEM((16, T128), jnp.float32),
            pltpu.VMEM((T128, FD), jnp.float32),
            pltpu.SemaphoreType.DMA,
        ],
    )
    def k(ft_hbm, idx_hbm, wc_hbm, out_hbm, idx_v, rows_v, wc_v, fv_v, sem):
        wid = lax.axis_index("s") * NC + lax.axis_index("c")
        ibase = wid * W2R
        pltpu.sync_copy(idx_hbm.at[pl.ds(ibase, W2R)], idx_v)

        def macro(m, carry):
            cps = []
            for u in range(16):
                cps.append(pltpu.async_copy(
                    ft_hbm.at[idx_v.at[m * 16 + u]],
                    rows_v.at[pl.ds(u * T128, T128)], sem))
            pltpu.sync_copy(wc_hbm.at[pl.ds(ibase + m * 16, 16)], wc_v)
            for c in cps:
                c.wait()

            def point(p, carry2):
                wvec = wc_v[p // 8, pl.ds((p % 8) * 16, 16)]
                base = p * 16
                acc = wvec[0] * rows_v[base, :]
                for j in range(1, 16):
                    acc = acc + wvec[j] * rows_v[base + j, :]
                fv_v[p, :] = acc
                return carry2

            lax.fori_loop(0, T128, point, 0)
            pltpu.sync_copy(
                fv_v,
                out_hbm.at[pl.ds(wid * (P // NW) + m * T128, T128),
                           pl.ds(0, FD)])
            return carry

        lax.fori_loop(0, W2R // 16, macro, 0)

    return k(ftab, idx, wcf)


_TC1_BLOCKS = 64
_PB = P // _TC1_BLOCKS
_RB = _PB * 16


def _tc1_body(fv_ref, enc_ref, sw1_ref, sw2_ref, cw1a_ref,
              cw1b_ref, cw2_ref, cw3_ref, fo_ref, col_ref):
    fvals = fv_ref[:, 0:FD]
    encr = enc_ref[...]
    enc = jnp.broadcast_to(encr[:, None, :],
                           (_PB // NI, NI, 16)).reshape(_PB, 16)
    h = jax.nn.relu(jnp.dot(fvals, sw1_ref[...], precision=_HI))
    fo = jnp.dot(h, sw2_ref[...], precision=_HI)
    c_pre = (jnp.dot(enc, cw1a_ref[...], precision=_HI)
             + jnp.dot(fo, cw1b_ref[...], precision=_HI))
    hc = jax.nn.relu(c_pre)
    hc2 = jax.nn.relu(jnp.dot(hc, cw2_ref[...], precision=_HI))
    col = jnp.dot(hc2, cw3_ref[...], precision=_HI)
    fo_ref[...] = fo
    col_ref[...] = col


def _tc1(fv, encr, sw1, sw2, cw1a, cw1b, cw2, cw3):
    rep2 = lambda shape: pl.BlockSpec(shape, lambda i: (0, 0))
    return pl.pallas_call(
        _tc1_body,
        grid=(_TC1_BLOCKS,),
        in_specs=[
            pl.BlockSpec((_PB, T128), lambda i: (i, 0)),
            pl.BlockSpec((_PB // NI, 16), lambda i: (i, 0)),
            rep2((16, 64)), rep2((64, 16)), rep2((16, 64)),
            rep2((16, 64)), rep2((64, 64)), rep2((64, 3)),
        ],
        out_specs=[
            pl.BlockSpec((_PB, 16), lambda i: (i, 0)),
            pl.BlockSpec((_PB, 3), lambda i: (i, 0)),
        ],
        out_shape=[
            jax.ShapeDtypeStruct((P, 16), jnp.float32),
            jax.ShapeDtypeStruct((P, 3), jnp.float32),
        ],
    )(fv, encr, sw1, sw2, cw1a, cw1b, cw2, cw3)


_TC2_BLOCKS = 32
_RAYB = B // _TC2_BLOCKS


def _tc2_body(sig_ref, mask_ref, c0_ref, c1_ref, c2_ref, out_ref):
    maskv = mask_ref[...]
    sigma = jnp.where(maskv > 0, sig_ref[...], 0.0)
    alpha = 1.0 - jnp.exp(-jax.nn.relu(sigma) * SS)
    x = 1.0 - alpha + 1e-10
    lx = jnp.log(x)
    r = lax.broadcasted_iota(jnp.int32, (NI, NI), 0)
    c = lax.broadcasted_iota(jnp.int32, (NI, NI), 1)
    m_excl = (r < c).astype(jnp.float32)
    t_tr = jnp.exp(jnp.dot(lx, m_excl, precision=_HI))
    al = alpha * t_tr
    bg = 1.0 - jnp.sum(al, axis=1, keepdims=True)
    outs = []
    for cref in (c0_ref, c1_ref, c2_ref):
        s = jax.nn.sigmoid(cref[...] * maskv)
        outs.append(jnp.sum(al * s, axis=1, keepdims=True) + bg)
    out_ref[...] = jnp.concatenate(outs, axis=1)


def _tc2(sig, maskf, c0, c1, c2):
    spec = pl.BlockSpec((_RAYB, NI), lambda i: (i, 0))
    return pl.pallas_call(
        _tc2_body,
        grid=(_TC2_BLOCKS,),
        in_specs=[spec, spec, spec, spec, spec],
        out_specs=pl.BlockSpec((_RAYB, 3), lambda i: (i, 0)),
        out_shape=jax.ShapeDtypeStruct((B, 3), jnp.float32),
    )(sig, maskf, c0, c1, c2)


def _tri_weights(o):
    x, y, z = o[:, 0], o[:, 1], o[:, 2]
    return jnp.stack([
        (1 - x) * (1 - y) * (1 - z),
        (1 - x) * (1 - y) * z,
        (1 - x) * y * (1 - z),
        (1 - x) * y * z,
        x * (1 - y) * (1 - z),
        x * (1 - y) * z,
        x * y * (1 - z),
        x * y * z,
    ], axis=-1)


def _sh16_enc(d):
    x, y, z = d[:, 0], d[:, 1], d[:, 2]
    xx, yy, zz = x * x, y * y, z * z
    return jnp.stack([
        0.28209479177387814 * jnp.ones_like(x),
        -0.48860251190291987 * y,
        0.48860251190291987 * z,
        -0.48860251190291987 * x,
        1.0925484305920792 * x * y,
        -1.0925484305920792 * y * z,
        0.31539156525252005 * (3.0 * zz - 1.0),
        -1.0925484305920792 * x * z,
        0.5462742152960396 * (xx - yy),
        -0.5900435899266435 * y * (3.0 * xx - yy),
        2.890611442640554 * x * y * z,
        -0.4570457994644658 * y * (4.0 * zz - xx - yy),
        0.3731763325901154 * z * (2.0 * zz - 3.0 * xx - 3.0 * yy),
        -0.4570457994644658 * x * (4.0 * zz - xx - yy),
        1.445305721320277 * z * (xx - yy),
        -0.5900435899266435 * x * (xx - 3.0 * yy),
    ], axis=-1)


def kernel(rays_o, rays_d, G, F, sigma_W1, sigma_W2, color_W1, color_W2,
           color_W3):
    rd = rays_d / jnp.linalg.norm(rays_d, axis=-1, keepdims=True)
    t = jnp.arange(NI + 1, dtype=jnp.float32) * SS
    tm = 0.5 * (t[:-1] + t[1:])

    px = (rays_o[:, 0:1] + rd[:, 0:1] * tm[None, :]).reshape(P)
    py = (rays_o[:, 1:2] + rd[:, 1:2] * tm[None, :]).reshape(P)
    pz = (rays_o[:, 2:3] + rd[:, 2:3] * tm[None, :]).reshape(P)
    mask = (jnp.sqrt(px * px + py * py + pz * pz) < RAD).reshape(B, NI)
    fpx = (px / RAD + 1.0) * (RES / 2.0)
    fpy = (py / RAD + 1.0) * (RES / 2.0)
    fpz = (pz / RAD + 1.0) * (RES / 2.0)

    def corners(fp):
        lo = jnp.clip(jnp.floor(fp - 0.5), 0.0, RES - 1)
        hi = jnp.clip(jnp.floor(fp + 0.5), 0.0, RES - 1)
        return lo, hi, fp - lo

    pfx0, pfx1, ox = corners(fpx)
    pfy0, pfy1, oy = corners(fpy)
    pfz0, pfz1, oz = corners(fpz)

    rep8 = lambda v: jnp.broadcast_to(v[:, None], (P, 8)).reshape(N1)
    j8 = jnp.arange(N1, dtype=jnp.int32) % 8
    xbit, ybit, zbit = j8 // 4, (j8 // 2) % 2, j8 % 2
    gx = jnp.where(xbit > 0, rep8(pfx1), rep8(pfx0))
    gy = jnp.where(ybit > 0, rep8(pfy1), rep8(pfy0))
    gz = jnp.where(zbit > 0, rep8(pfz1), rep8(pfz0))
    gidxf = (gx.astype(jnp.int32) * (RES * RES)
             + gy.astype(jnp.int32) * RES + gz.astype(jnp.int32))
    fx = jnp.where(xbit > 0, rep8(ox), 1.0 - rep8(ox))
    fy = jnp.where(ybit > 0, rep8(oy), 1.0 - rep8(oy))
    fz = jnp.where(zbit > 0, rep8(oz), 1.0 - rep8(oz))
    w8f = fx * fy * fz

    gvf = _sc_gather_scalar(G.reshape(-1),
                            gidxf.reshape(N1R, T128)).reshape(N1)

    fi = (jnp.clip(gvf, -1.0, 1.0) + 1.0) * (NF / 2.0)
    fl = jnp.clip(jnp.floor(fi), 0.0, NF - 1)
    ce = jnp.clip(jnp.ceil(fi), 0.0, NF - 1)
    wlo = w8f * (ce - fi)
    whi = w8f * (fi - fl)
    rep2 = lambda v: jnp.broadcast_to(v[:, None], (N1, 2)).reshape(N2)
    par = jnp.arange(N2, dtype=jnp.int32) % 2
    wcf = jnp.where(par > 0, rep2(whi), rep2(wlo)).reshape(N2R, T128)
    idx2 = jnp.where(par > 0, rep2(ce), rep2(fl)).astype(
        jnp.int32).reshape(N2R, T128)

    fv = _sc_gather_reduce(F, idx2, wcf)

    enc = _sh16_enc(rd)
    cw1b = jnp.concatenate([jnp.zeros((1, 64), jnp.float32), color_W1[16:]],
                           axis=0)
    fo, col = _tc1(fv, enc, sigma_W1, sigma_W2, color_W1[:16],
                   cw1b, color_W2, color_W3)

    maskf = mask.astype(jnp.float32)
    sig = fo[:, 0].reshape(B, NI)
    c0 = col[:, 0].reshape(B, NI)
    c1 = col[:, 1].reshape(B, NI)
    c2 = col[:, 2].reshape(B, NI)
    return _tc2(sig, maskf, c0, c1, c2)


# speedup vs baseline: 3.1586x; 1.4068x over previous
import functools

import jax
import jax.numpy as jnp
from jax import lax
from jax.experimental import pallas as pl
from jax.experimental.pallas import tpu as pltpu
from jax.experimental.pallas import tpu_sc as plsc

RES = 256
NF = 1048576
FD = 16
RAD = 1.0
NI = 32
SS = 0.0625

B = 4096
P = B * NI
N1 = P * 8
N2 = P * 16
NC, NS = 2, 16
NW = NC * NS
T128 = 128

N1R = N1 // T128
N2R = N2 // T128
W1R = N1R // NW
W2R = N2R // NW

_HI = jax.lax.Precision.HIGHEST
_HP = jax.lax.Precision.DEFAULT


def _sc_gather_scalar(gtab, idx):
    mesh = plsc.VectorSubcoreMesh(core_axis_name="c", subcore_axis_name="s")

    @functools.partial(
        pl.kernel, mesh=mesh,
        out_type=jax.ShapeDtypeStruct((N1R, T128), jnp.float32),
        compiler_params=pltpu.CompilerParams(use_tc_tiling_on_sc=False),
        scratch_types=[
            pltpu.VMEM((W1R, T128), jnp.int32),
            pltpu.VMEM((W1R, T128), jnp.float32),
            pltpu.SemaphoreType.DMA,
        ],
    )
    def k(gt_hbm, idx_hbm, out_hbm, idx_v, out_v, sem):
        wid = lax.axis_index("s") * NC + lax.axis_index("c")
        base = wid * W1R
        pltpu.sync_copy(idx_hbm.at[pl.ds(base, W1R)], idx_v)

        def macro(m, carry):
            cps = []
            for u in range(16):
                j = m * 16 + u
                cps.append(pltpu.async_copy(
                    gt_hbm.at[idx_v.at[j]], out_v.at[j], sem))
            for c in cps:
                c.wait()
            return carry

        lax.fori_loop(0, W1R // 16, macro, 0)
        pltpu.sync_copy(out_v, out_hbm.at[pl.ds(base, W1R)])

    return k(gtab, idx)


def _sc_gather_reduce(ftab, idx, wcf):
    mesh = plsc.VectorSubcoreMesh(core_axis_name="c", subcore_axis_name="s")

    @functools.partial(
        pl.kernel, mesh=mesh,
        out_type=jax.ShapeDtypeStruct((P, T128), jnp.float32),
        compiler_params=pltpu.CompilerParams(use_tc_tiling_on_sc=False),
        scratch_types=[
            pltpu.VMEM((W2R, T128), jnp.int32),
            pltpu.VMEM((16 * T128, FD), jnp.float32),
            pltpu.VMEM((16, T128), jnp.float32),
            pltpu.VMEM((T128, FD), jnp.float32),
            pltpu.SemaphoreType.DMA,
        ],
    )
    def k(ft_hbm, idx_hbm, wc_hbm, out_hbm, idx_v, rows_v, wc_v, fv_v, sem):
        wid = lax.axis_index("s") * NC + lax.axis_index("c")
        ibase = wid * W2R
        pltpu.sync_copy(idx_hbm.at[pl.ds(ibase, W2R)], idx_v)

        def macro(m, carry):
            cps = []
            for u in range(16):
                cps.append(pltpu.async_copy(
                    ft_hbm.at[idx_v.at[m * 16 + u]],
                    rows_v.at[pl.ds(u * T128, T128)], sem))
            pltpu.sync_copy(wc_hbm.at[pl.ds(ibase + m * 16, 16)], wc_v)
            for c in cps:
                c.wait()

            def point(p, carry2):
                wvec = wc_v[p // 8, pl.ds((p % 8) * 16, 16)]
                base = p * 16
                acc = wvec[0] * rows_v[base, :]
                for j in range(1, 16):
                    acc = acc + wvec[j] * rows_v[base + j, :]
                fv_v[p, :] = acc
                return carry2

            lax.fori_loop(0, T128, point, 0)
            pltpu.sync_copy(
                fv_v,
                out_hbm.at[pl.ds(wid * (P // NW) + m * T128, T128),
                           pl.ds(0, FD)])
            return carry

        lax.fori_loop(0, W2R // 16, macro, 0)

    return k(ftab, idx, wcf)


_TC1_BLOCKS = 32
_PB = P // _TC1_BLOCKS
_RB = _PB * 16


def _tc1_body(fv_ref, enc_ref, sw1_ref, sw2_ref, cw1a_ref,
              cw1b_ref, cw2_ref, cw3_ref, fo_ref, col_ref):
    fvals = fv_ref[:, 0:FD]
    encr = enc_ref[...]
    enc = jnp.broadcast_to(encr[:, None, :],
                           (_PB // NI, NI, 16)).reshape(_PB, 16)
    h = jax.nn.relu(jnp.dot(fvals, sw1_ref[...], precision=_HP))
    fo = jnp.dot(h, sw2_ref[...], precision=_HP)
    c_pre = (jnp.dot(enc, cw1a_ref[...], precision=_HP)
             + jnp.dot(fo, cw1b_ref[...], precision=_HP))
    hc = jax.nn.relu(c_pre)
    hc2 = jax.nn.relu(jnp.dot(hc, cw2_ref[...], precision=_HP))
    col = jnp.dot(hc2, cw3_ref[...], precision=_HP)
    fo_ref[...] = fo
    col_ref[...] = col


def _tc1(fv, encr, sw1, sw2, cw1a, cw1b, cw2, cw3):
    rep2 = lambda shape: pl.BlockSpec(shape, lambda i: (0, 0))
    return pl.pallas_call(
        _tc1_body,
        grid=(_TC1_BLOCKS,),
        in_specs=[
            pl.BlockSpec((_PB, T128), lambda i: (i, 0)),
            pl.BlockSpec((_PB // NI, 16), lambda i: (i, 0)),
            rep2((16, 64)), rep2((64, 16)), rep2((16, 64)),
            rep2((16, 64)), rep2((64, 64)), rep2((64, 3)),
        ],
        out_specs=[
            pl.BlockSpec((_PB, 16), lambda i: (i, 0)),
            pl.BlockSpec((_PB, 3), lambda i: (i, 0)),
        ],
        out_shape=[
            jax.ShapeDtypeStruct((P, 16), jnp.float32),
            jax.ShapeDtypeStruct((P, 3), jnp.float32),
        ],
    )(fv, encr, sw1, sw2, cw1a, cw1b, cw2, cw3)


_TC2_BLOCKS = 32
_RAYB = B // _TC2_BLOCKS


def _tc2_body(sig_ref, mask_ref, c0_ref, c1_ref, c2_ref, out_ref):
    maskv = mask_ref[...]
    sigma = jnp.where(maskv > 0, sig_ref[...], 0.0)
    alpha = 1.0 - jnp.exp(-jax.nn.relu(sigma) * SS)
    x = 1.0 - alpha + 1e-10
    lx = jnp.log(x)
    r = lax.broadcasted_iota(jnp.int32, (NI, NI), 0)
    c = lax.broadcasted_iota(jnp.int32, (NI, NI), 1)
    m_excl = (r < c).astype(jnp.float32)
    t_tr = jnp.exp(jnp.dot(lx, m_excl, precision=_HI))
    al = alpha * t_tr
    bg = 1.0 - jnp.sum(al, axis=1, keepdims=True)
    outs = []
    for cref in (c0_ref, c1_ref, c2_ref):
        s = jax.nn.sigmoid(cref[...] * maskv)
        outs.append(jnp.sum(al * s, axis=1, keepdims=True) + bg)
    out_ref[...] = jnp.concatenate(outs, axis=1)


def _tc2(sig, maskf, c0, c1, c2):
    spec = pl.BlockSpec((_RAYB, NI), lambda i: (i, 0))
    return pl.pallas_call(
        _tc2_body,
        grid=(_TC2_BLOCKS,),
        in_specs=[spec, spec, spec, spec, spec],
        out_specs=pl.BlockSpec((_RAYB, 3), lambda i: (i, 0)),
        out_shape=jax.ShapeDtypeStruct((B, 3), jnp.float32),
    )(sig, maskf, c0, c1, c2)


def _tri_weights(o):
    x, y, z = o[:, 0], o[:, 1], o[:, 2]
    return jnp.stack([
        (1 - x) * (1 - y) * (1 - z),
        (1 - x) * (1 - y) * z,
        (1 - x) * y * (1 - z),
        (1 - x) * y * z,
        x * (1 - y) * (1 - z),
        x * (1 - y) * z,
        x * y * (1 - z),
        x * y * z,
    ], axis=-1)


def _sh16_enc(d):
    x, y, z = d[:, 0], d[:, 1], d[:, 2]
    xx, yy, zz = x * x, y * y, z * z
    return jnp.stack([
        0.28209479177387814 * jnp.ones_like(x),
        -0.48860251190291987 * y,
        0.48860251190291987 * z,
        -0.48860251190291987 * x,
        1.0925484305920792 * x * y,
        -1.0925484305920792 * y * z,
        0.31539156525252005 * (3.0 * zz - 1.0),
        -1.0925484305920792 * x * z,
        0.5462742152960396 * (xx - yy),
        -0.5900435899266435 * y * (3.0 * xx - yy),
        2.890611442640554 * x * y * z,
        -0.4570457994644658 * y * (4.0 * zz - xx - yy),
        0.3731763325901154 * z * (2.0 * zz - 3.0 * xx - 3.0 * yy),
        -0.4570457994644658 * x * (4.0 * zz - xx - yy),
        1.445305721320277 * z * (xx - yy),
        -0.5900435899266435 * x * (xx - 3.0 * yy),
    ], axis=-1)


def kernel(rays_o, rays_d, G, F, sigma_W1, sigma_W2, color_W1, color_W2,
           color_W3):
    rd = rays_d / jnp.linalg.norm(rays_d, axis=-1, keepdims=True)
    t = jnp.arange(NI + 1, dtype=jnp.float32) * SS
    tm = 0.5 * (t[:-1] + t[1:])

    px = (rays_o[:, 0:1] + rd[:, 0:1] * tm[None, :]).reshape(P)
    py = (rays_o[:, 1:2] + rd[:, 1:2] * tm[None, :]).reshape(P)
    pz = (rays_o[:, 2:3] + rd[:, 2:3] * tm[None, :]).reshape(P)
    mask = (jnp.sqrt(px * px + py * py + pz * pz) < RAD).reshape(B, NI)
    fpx = (px / RAD + 1.0) * (RES / 2.0)
    fpy = (py / RAD + 1.0) * (RES / 2.0)
    fpz = (pz / RAD + 1.0) * (RES / 2.0)

    def corners(fp):
        lo = jnp.clip(jnp.floor(fp - 0.5), 0.0, RES - 1)
        hi = jnp.clip(jnp.floor(fp + 0.5), 0.0, RES - 1)
        return lo, hi, fp - lo

    pfx0, pfx1, ox = corners(fpx)
    pfy0, pfy1, oy = corners(fpy)
    pfz0, pfz1, oz = corners(fpz)

    rep8 = lambda v: jnp.broadcast_to(v[:, None], (P, 8)).reshape(N1)
    j8 = jnp.arange(N1, dtype=jnp.int32) % 8
    xbit, ybit, zbit = j8 // 4, (j8 // 2) % 2, j8 % 2
    gx = jnp.where(xbit > 0, rep8(pfx1), rep8(pfx0))
    gy = jnp.where(ybit > 0, rep8(pfy1), rep8(pfy0))
    gz = jnp.where(zbit > 0, rep8(pfz1), rep8(pfz0))
    gidxf = (gx.astype(jnp.int32) * (RES * RES)
             + gy.astype(jnp.int32) * RES + gz.astype(jnp.int32))
    fx = jnp.where(xbit > 0, rep8(ox), 1.0 - rep8(ox))
    fy = jnp.where(ybit > 0, rep8(oy), 1.0 - rep8(oy))
    fz = jnp.where(zbit > 0, rep8(oz), 1.0 - rep8(oz))
    w8f = fx * fy * fz

    gvf = _sc_gather_scalar(G.reshape(-1),
                            gidxf.reshape(N1R, T128)).reshape(N1)

    fi = (jnp.clip(gvf, -1.0, 1.0) + 1.0) * (NF / 2.0)
    fl = jnp.clip(jnp.floor(fi), 0.0, NF - 1)
    ce = jnp.clip(jnp.ceil(fi), 0.0, NF - 1)
    wlo = w8f * (ce - fi)
    whi = w8f * (fi - fl)
    rep2 = lambda v: jnp.broadcast_to(v[:, None], (N1, 2)).reshape(N2)
    par = jnp.arange(N2, dtype=jnp.int32) % 2
    wcf = jnp.where(par > 0, rep2(whi), rep2(wlo)).reshape(N2R, T128)
    idx2 = jnp.where(par > 0, rep2(ce), rep2(fl)).astype(
        jnp.int32).reshape(N2R, T128)

    fv = _sc_gather_reduce(F, idx2, wcf)

    enc = _sh16_enc(rd)
    cw1b = jnp.concatenate([jnp.zeros((1, 64), jnp.float32), color_W1[16:]],
                           axis=0)
    fo, col = _tc1(fv, enc, sigma_W1, sigma_W2, color_W1[:16],
                   cw1b, color_W2, color_W3)

    maskf = mask.astype(jnp.float32)
    sig = fo[:, 0].reshape(B, NI)
    c0 = col[:, 0].reshape(B, NI)
    c1 = col[:, 1].reshape(B, NI)
    c2 = col[:, 2].reshape(B, NI)
    return _tc2(sig, maskf, c0, c1, c2)


# speedup vs baseline: 3.3398x; 1.0574x over previous
import functools

import jax
import jax.numpy as jnp
from jax import lax
from jax.experimental import pallas as pl
from jax.experimental.pallas import tpu as pltpu
from jax.experimental.pallas import tpu_sc as plsc

RES = 256
NF = 1048576
FD = 16
RAD = 1.0
NI = 32
SS = 0.0625

B = 4096
P = B * NI
N1 = P * 8
N2 = P * 16
NC, NS = 2, 16
NW = NC * NS
T128 = 128

N1R = N1 // T128
N2R = N2 // T128
W1R = N1R // NW
W2R = N2R // NW

_HI = jax.lax.Precision.HIGHEST
_HP = jax.lax.Precision.DEFAULT


def _sc_gather_scalar(gtab, idx):
    mesh = plsc.VectorSubcoreMesh(core_axis_name="c", subcore_axis_name="s")

    @functools.partial(
        pl.kernel, mesh=mesh,
        out_type=jax.ShapeDtypeStruct((N1R, T128), jnp.float32),
        compiler_params=pltpu.CompilerParams(use_tc_tiling_on_sc=False),
        scratch_types=[
            pltpu.VMEM((W1R, T128), jnp.int32),
            pltpu.VMEM((W1R, T128), jnp.float32),
            pltpu.SemaphoreType.DMA,
        ],
    )
    def k(gt_hbm, idx_hbm, out_hbm, idx_v, out_v, sem):
        wid = lax.axis_index("s") * NC + lax.axis_index("c")
        base = wid * W1R
        pltpu.sync_copy(idx_hbm.at[pl.ds(base, W1R)], idx_v)

        def macro(m, carry):
            cps = []
            for u in range(16):
                j = m * 16 + u
                cps.append(pltpu.async_copy(
                    gt_hbm.at[idx_v.at[j]], out_v.at[j], sem))
            for c in cps:
                c.wait()
            return carry

        lax.fori_loop(0, W1R // 16, macro, 0)
        pltpu.sync_copy(out_v, out_hbm.at[pl.ds(base, W1R)])

    return k(gtab, idx)


def _sc_gather_reduce(ftab, idx, wcf):
    mesh = plsc.VectorSubcoreMesh(core_axis_name="c", subcore_axis_name="s")

    @functools.partial(
        pl.kernel, mesh=mesh,
        out_type=jax.ShapeDtypeStruct((P, T128), jnp.float32),
        compiler_params=pltpu.CompilerParams(use_tc_tiling_on_sc=False),
        scratch_types=[
            pltpu.VMEM((W2R, T128), jnp.int32),
            pltpu.VMEM((16 * T128, FD), jnp.float32),
            pltpu.VMEM((16, T128), jnp.float32),
            pltpu.VMEM((T128, FD), jnp.float32),
            pltpu.SemaphoreType.DMA,
        ],
    )
    def k(ft_hbm, idx_hbm, wc_hbm, out_hbm, idx_v, rows_v, wc_v, fv_v, sem):
        wid = lax.axis_index("s") * NC + lax.axis_index("c")
        ibase = wid * W2R
        pltpu.sync_copy(idx_hbm.at[pl.ds(ibase, W2R)], idx_v)

        def macro(m, carry):
            cps = []
            for u in range(16):
                cps.append(pltpu.async_copy(
                    ft_hbm.at[idx_v.at[m * 16 + u]],
                    rows_v.at[pl.ds(u * T128, T128)], sem))
            pltpu.sync_copy(wc_hbm.at[pl.ds(ibase + m * 16, 16)], wc_v)
            for c in cps:
                c.wait()

            def point(p, carry2):
                wvec = wc_v[p // 8, pl.ds((p % 8) * 16, 16)]
                base = p * 16
                acc = wvec[0] * rows_v[base, :]
                for j in range(1, 16):
                    acc = acc + wvec[j] * rows_v[base + j, :]
                fv_v[p, :] = acc
                return carry2

            lax.fori_loop(0, T128, point, 0)
            pltpu.sync_copy(
                fv_v,
                out_hbm.at[pl.ds(wid * (P // NW) + m * T128, T128),
                           pl.ds(0, FD)])
            return carry

        lax.fori_loop(0, W2R // 16, macro, 0)

    return k(ftab, idx, wcf)


_TC1_BLOCKS = 32
_PB = P // _TC1_BLOCKS
_RB = _PB * 16


def _tc1_body(fv_ref, enc_ref, mask_ref, sw1_ref, sw2_ref, cw1a_ref,
              cw1b_ref, cw2_ref, cw3_ref, out_ref):
    fvals = fv_ref[:, 0:FD]
    encr = enc_ref[...]
    enc = jnp.broadcast_to(encr[:, None, :],
                           (_PB // NI, NI, 16)).reshape(_PB, 16)
    h = jax.nn.relu(jnp.dot(fvals, sw1_ref[...], precision=_HP))
    fo = jnp.dot(h, sw2_ref[...], precision=_HP)
    c_pre = (jnp.dot(enc, cw1a_ref[...], precision=_HP)
             + jnp.dot(fo, cw1b_ref[...], precision=_HP))
    hc = jax.nn.relu(c_pre)
    hc2 = jax.nn.relu(jnp.dot(hc, cw2_ref[...], precision=_HP))
    col = jnp.dot(hc2, cw3_ref[...], precision=_HP)

    nr = _PB // NI
    maskv = mask_ref[...]
    sig = fo[:, 0].reshape(nr, NI)
    sigma = jnp.where(maskv > 0, sig, 0.0)
    alpha = 1.0 - jnp.exp(-jax.nn.relu(sigma) * SS)
    x = 1.0 - alpha + 1e-10
    lx = jnp.log(x)
    r = lax.broadcasted_iota(jnp.int32, (NI, NI), 0)
    c = lax.broadcasted_iota(jnp.int32, (NI, NI), 1)
    m_excl = (r < c).astype(jnp.float32)
    t_tr = jnp.exp(jnp.dot(lx, m_excl, precision=_HI))
    al = alpha * t_tr
    bg = 1.0 - jnp.sum(al, axis=1, keepdims=True)
    outs = []
    for k in range(3):
        s = jax.nn.sigmoid(col[:, k].reshape(nr, NI) * maskv)
        outs.append(jnp.sum(al * s, axis=1, keepdims=True) + bg)
    out_ref[...] = jnp.concatenate(outs, axis=1)


def _tc1(fv, encr, maskf, sw1, sw2, cw1a, cw1b, cw2, cw3):
    rep2 = lambda shape: pl.BlockSpec(shape, lambda i: (0, 0))
    return pl.pallas_call(
        _tc1_body,
        grid=(_TC1_BLOCKS,),
        in_specs=[
            pl.BlockSpec((_PB, T128), lambda i: (i, 0)),
            pl.BlockSpec((_PB // NI, 16), lambda i: (i, 0)),
            pl.BlockSpec((_PB // NI, NI), lambda i: (i, 0)),
            rep2((16, 64)), rep2((64, 16)), rep2((16, 64)),
            rep2((16, 64)), rep2((64, 64)), rep2((64, 3)),
        ],
        out_specs=pl.BlockSpec((_PB // NI, 3), lambda i: (i, 0)),
        out_shape=jax.ShapeDtypeStruct((B, 3), jnp.float32),
    )(fv, encr, maskf, sw1, sw2, cw1a, cw1b, cw2, cw3)


def _tri_weights(o):
    x, y, z = o[:, 0], o[:, 1], o[:, 2]
    return jnp.stack([
        (1 - x) * (1 - y) * (1 - z),
        (1 - x) * (1 - y) * z,
        (1 - x) * y * (1 - z),
        (1 - x) * y * z,
        x * (1 - y) * (1 - z),
        x * (1 - y) * z,
        x * y * (1 - z),
        x * y * z,
    ], axis=-1)


def _sh16_enc(d):
    x, y, z = d[:, 0], d[:, 1], d[:, 2]
    xx, yy, zz = x * x, y * y, z * z
    return jnp.stack([
        0.28209479177387814 * jnp.ones_like(x),
        -0.48860251190291987 * y,
        0.48860251190291987 * z,
        -0.48860251190291987 * x,
        1.0925484305920792 * x * y,
        -1.0925484305920792 * y * z,
        0.31539156525252005 * (3.0 * zz - 1.0),
        -1.0925484305920792 * x * z,
        0.5462742152960396 * (xx - yy),
        -0.5900435899266435 * y * (3.0 * xx - yy),
        2.890611442640554 * x * y * z,
        -0.4570457994644658 * y * (4.0 * zz - xx - yy),
        0.3731763325901154 * z * (2.0 * zz - 3.0 * xx - 3.0 * yy),
        -0.4570457994644658 * x * (4.0 * zz - xx - yy),
        1.445305721320277 * z * (xx - yy),
        -0.5900435899266435 * x * (xx - 3.0 * yy),
    ], axis=-1)


def kernel(rays_o, rays_d, G, F, sigma_W1, sigma_W2, color_W1, color_W2,
           color_W3):
    rd = rays_d / jnp.linalg.norm(rays_d, axis=-1, keepdims=True)
    t = jnp.arange(NI + 1, dtype=jnp.float32) * SS
    tm = 0.5 * (t[:-1] + t[1:])

    px = (rays_o[:, 0:1] + rd[:, 0:1] * tm[None, :]).reshape(P)
    py = (rays_o[:, 1:2] + rd[:, 1:2] * tm[None, :]).reshape(P)
    pz = (rays_o[:, 2:3] + rd[:, 2:3] * tm[None, :]).reshape(P)
    mask = (jnp.sqrt(px * px + py * py + pz * pz) < RAD).reshape(B, NI)
    fpx = (px / RAD + 1.0) * (RES / 2.0)
    fpy = (py / RAD + 1.0) * (RES / 2.0)
    fpz = (pz / RAD + 1.0) * (RES / 2.0)

    def corners(fp):
        lo = jnp.clip(jnp.floor(fp - 0.5), 0.0, RES - 1)
        hi = jnp.clip(jnp.floor(fp + 0.5), 0.0, RES - 1)
        return lo, hi, fp - lo

    pfx0, pfx1, ox = corners(fpx)
    pfy0, pfy1, oy = corners(fpy)
    pfz0, pfz1, oz = corners(fpz)

    rep8 = lambda v: jnp.broadcast_to(v[:, None], (P, 8)).reshape(N1)
    j8 = jnp.arange(N1, dtype=jnp.int32) % 8
    xbit, ybit, zbit = j8 // 4, (j8 // 2) % 2, j8 % 2
    gx = jnp.where(xbit > 0, rep8(pfx1), rep8(pfx0))
    gy = jnp.where(ybit > 0, rep8(pfy1), rep8(pfy0))
    gz = jnp.where(zbit > 0, rep8(pfz1), rep8(pfz0))
    gidxf = (gx.astype(jnp.int32) * (RES * RES)
             + gy.astype(jnp.int32) * RES + gz.astype(jnp.int32))
    fx = jnp.where(xbit > 0, rep8(ox), 1.0 - rep8(ox))
    fy = jnp.where(ybit > 0, rep8(oy), 1.0 - rep8(oy))
    fz = jnp.where(zbit > 0, rep8(oz), 1.0 - rep8(oz))
    w8f = fx * fy * fz

    gvf = _sc_gather_scalar(G.reshape(-1),
                            gidxf.reshape(N1R, T128)).reshape(N1)

    fi = (jnp.clip(gvf, -1.0, 1.0) + 1.0) * (NF / 2.0)
    fl = jnp.clip(jnp.floor(fi), 0.0, NF - 1)
    ce = jnp.clip(jnp.ceil(fi), 0.0, NF - 1)
    wlo = w8f * (ce - fi)
    whi = w8f * (fi - fl)
    rep2 = lambda v: jnp.broadcast_to(v[:, None], (N1, 2)).reshape(N2)
    par = jnp.arange(N2, dtype=jnp.int32) % 2
    wcf = jnp.where(par > 0, rep2(whi), rep2(wlo)).reshape(N2R, T128)
    idx2 = jnp.where(par > 0, rep2(ce), rep2(fl)).astype(
        jnp.int32).reshape(N2R, T128)

    fv = _sc_gather_reduce(F, idx2, wcf)

    enc = _sh16_enc(rd)
    cw1b = jnp.concatenate([jnp.zeros((1, 64), jnp.float32), color_W1[16:]],
                           axis=0)
    maskf = mask.astype(jnp.float32)
    return _tc1(fv, enc, maskf, sigma_W1, sigma_W2, color_W1[:16],
                cw1b, color_W2, color_W3)


# speedup vs baseline: 3.4148x; 1.0225x over previous
import functools

import jax
import jax.numpy as jnp
from jax import lax
from jax.experimental import pallas as pl
from jax.experimental.pallas import tpu as pltpu
from jax.experimental.pallas import tpu_sc as plsc

RES = 256
NF = 1048576
FD = 16
RAD = 1.0
NI = 32
SS = 0.0625

B = 4096
P = B * NI
N1 = P * 8
N2 = P * 16
NC, NS = 2, 16
NW = NC * NS
T128 = 128

N1R = N1 // T128
N2R = N2 // T128
W1R = N1R // NW
W2R = N2R // NW

_HI = jax.lax.Precision.HIGHEST
_HP = jax.lax.Precision.DEFAULT


def _sc_gather_scalar(gtab, idx):
    mesh = plsc.VectorSubcoreMesh(core_axis_name="c", subcore_axis_name="s")

    @functools.partial(
        pl.kernel, mesh=mesh,
        out_type=jax.ShapeDtypeStruct((N1R, T128), jnp.float32),
        compiler_params=pltpu.CompilerParams(use_tc_tiling_on_sc=False),
        scratch_types=[
            pltpu.VMEM((W1R, T128), jnp.int32),
            pltpu.VMEM((W1R, T128), jnp.float32),
            pltpu.SemaphoreType.DMA,
        ],
    )
    def k(gt_hbm, idx_hbm, out_hbm, idx_v, out_v, sem):
        wid = lax.axis_index("s") * NC + lax.axis_index("c")
        base = wid * W1R
        pltpu.sync_copy(idx_hbm.at[pl.ds(base, W1R)], idx_v)

        def fire(m):
            for u in range(16):
                j = m * 16 + u
                pltpu.async_copy(gt_hbm.at[idx_v.at[j]], out_v.at[j], sem)

        fire(0)

        def macro(m, carry):
            @pl.when(m + 1 < W1R // 16)
            def _():
                for u in range(16):
                    j = (m + 1) * 16 + u
                    pltpu.async_copy(gt_hbm.at[idx_v.at[j]], out_v.at[j], sem)

            pltpu.make_async_copy(
                out_hbm.at[pl.ds(0, 16)],
                out_v.at[pl.ds(m * 16, 16)], sem).wait()
            return carry

        lax.fori_loop(0, W1R // 16, macro, 0)
        pltpu.sync_copy(out_v, out_hbm.at[pl.ds(base, W1R)])

    return k(gtab, idx)


def _sc_gather_reduce(ftab, idx, wcf):
    mesh = plsc.VectorSubcoreMesh(core_axis_name="c", subcore_axis_name="s")

    @functools.partial(
        pl.kernel, mesh=mesh,
        out_type=jax.ShapeDtypeStruct((P, T128), jnp.float32),
        compiler_params=pltpu.CompilerParams(use_tc_tiling_on_sc=False),
        scratch_types=[
            pltpu.VMEM((W2R, T128), jnp.int32),
            pltpu.VMEM((16 * T128, FD), jnp.float32),
            pltpu.VMEM((8, T128), jnp.float32),
            pltpu.VMEM((64, FD), jnp.float32),
            pltpu.SemaphoreType.DMA,
        ],
    )
    def k(ft_hbm, idx_hbm, wc_hbm, out_hbm, idx_v, rows_v, wc_v, fv_v, sem):
        wid = lax.axis_index("s") * NC + lax.axis_index("c")
        ibase = wid * W2R
        pltpu.sync_copy(idx_hbm.at[pl.ds(ibase, W2R)], idx_v)
        nm = W2R // 8

        def fire(m, boff):
            for u in range(8):
                pltpu.async_copy(
                    ft_hbm.at[idx_v.at[m * 8 + u]],
                    rows_v.at[pl.ds(boff + u * T128, T128)], sem)

        fire(0, 0)

        def macro(m, carry):
            boff = (m % 2) * (8 * T128)

            @pl.when(m + 1 < nm)
            def _():
                fire(m + 1, ((m + 1) % 2) * (8 * T128))

            pltpu.sync_copy(wc_hbm.at[pl.ds(ibase + m * 8, 8)], wc_v)
            pltpu.make_async_copy(
                ft_hbm.at[pl.ds(0, 8 * T128)],
                rows_v.at[pl.ds(boff, 8 * T128)], sem).wait()

            def point(p, carry2):
                wvec = wc_v[p // 8, pl.ds((p % 8) * 16, 16)]
                base = boff + p * 16
                acc = wvec[0] * rows_v[base, :]
                for j in range(1, 16):
                    acc = acc + wvec[j] * rows_v[base + j, :]
                fv_v[p, :] = acc
                return carry2

            lax.fori_loop(0, 64, point, 0)
            pltpu.sync_copy(
                fv_v,
                out_hbm.at[pl.ds(wid * (P // NW) + m * 64, 64),
                           pl.ds(0, FD)])
            return carry

        lax.fori_loop(0, nm, macro, 0)

    return k(ftab, idx, wcf)


_TC1_BLOCKS = 32
_PB = P // _TC1_BLOCKS
_RB = _PB * 16


def _tc1_body(fv_ref, enc_ref, mask_ref, sw1_ref, sw2_ref, cw1a_ref,
              cw1b_ref, cw2_ref, cw3_ref, out_ref):
    fvals = fv_ref[:, 0:FD]
    encr = enc_ref[...]
    enc = jnp.broadcast_to(encr[:, None, :],
                           (_PB // NI, NI, 16)).reshape(_PB, 16)
    h = jax.nn.relu(jnp.dot(fvals, sw1_ref[...], precision=_HP))
    fo = jnp.dot(h, sw2_ref[...], precision=_HP)
    c_pre = (jnp.dot(enc, cw1a_ref[...], precision=_HP)
             + jnp.dot(fo, cw1b_ref[...], precision=_HP))
    hc = jax.nn.relu(c_pre)
    hc2 = jax.nn.relu(jnp.dot(hc, cw2_ref[...], precision=_HP))
    col = jnp.dot(hc2, cw3_ref[...], precision=_HP)

    nr = _PB // NI
    maskv = mask_ref[...]
    sig = fo[:, 0].reshape(nr, NI)
    sigma = jnp.where(maskv > 0, sig, 0.0)
    alpha = 1.0 - jnp.exp(-jax.nn.relu(sigma) * SS)
    x = 1.0 - alpha + 1e-10
    lx = jnp.log(x)
    r = lax.broadcasted_iota(jnp.int32, (NI, NI), 0)
    c = lax.broadcasted_iota(jnp.int32, (NI, NI), 1)
    m_excl = (r < c).astype(jnp.float32)
    t_tr = jnp.exp(jnp.dot(lx, m_excl, precision=_HI))
    al = alpha * t_tr
    bg = 1.0 - jnp.sum(al, axis=1, keepdims=True)
    outs = []
    for k in range(3):
        s = jax.nn.sigmoid(col[:, k].reshape(nr, NI) * maskv)
        outs.append(jnp.sum(al * s, axis=1, keepdims=True) + bg)
    out_ref[...] = jnp.concatenate(outs, axis=1)


def _tc1(fv, encr, maskf, sw1, sw2, cw1a, cw1b, cw2, cw3):
    rep2 = lambda shape: pl.BlockSpec(shape, lambda i: (0, 0))
    return pl.pallas_call(
        _tc1_body,
        grid=(_TC1_BLOCKS,),
        in_specs=[
            pl.BlockSpec((_PB, T128), lambda i: (i, 0)),
            pl.BlockSpec((_PB // NI, 16), lambda i: (i, 0)),
            pl.BlockSpec((_PB // NI, NI), lambda i: (i, 0)),
            rep2((16, 64)), rep2((64, 16)), rep2((16, 64)),
            rep2((16, 64)), rep2((64, 64)), rep2((64, 3)),
        ],
        out_specs=pl.BlockSpec((_PB // NI, 3), lambda i: (i, 0)),
        out_shape=jax.ShapeDtypeStruct((B, 3), jnp.float32),
    )(fv, encr, maskf, sw1, sw2, cw1a, cw1b, cw2, cw3)


def _tri_weights(o):
    x, y, z = o[:, 0], o[:, 1], o[:, 2]
    return jnp.stack([
        (1 - x) * (1 - y) * (1 - z),
        (1 - x) * (1 - y) * z,
        (1 - x) * y * (1 - z),
        (1 - x) * y * z,
        x * (1 - y) * (1 - z),
        x * (1 - y) * z,
        x * y * (1 - z),
        x * y * z,
    ], axis=-1)


def _sh16_enc(d):
    x, y, z = d[:, 0], d[:, 1], d[:, 2]
    xx, yy, zz = x * x, y * y, z * z
    return jnp.stack([
        0.28209479177387814 * jnp.ones_like(x),
        -0.48860251190291987 * y,
        0.48860251190291987 * z,
        -0.48860251190291987 * x,
        1.0925484305920792 * x * y,
        -1.0925484305920792 * y * z,
        0.31539156525252005 * (3.0 * zz - 1.0),
        -1.0925484305920792 * x * z,
        0.5462742152960396 * (xx - yy),
        -0.5900435899266435 * y * (3.0 * xx - yy),
        2.890611442640554 * x * y * z,
        -0.4570457994644658 * y * (4.0 * zz - xx - yy),
        0.3731763325901154 * z * (2.0 * zz - 3.0 * xx - 3.0 * yy),
        -0.4570457994644658 * x * (4.0 * zz - xx - yy),
        1.445305721320277 * z * (xx - yy),
        -0.5900435899266435 * x * (xx - 3.0 * yy),
    ], axis=-1)


def kernel(rays_o, rays_d, G, F, sigma_W1, sigma_W2, color_W1, color_W2,
           color_W3):
    rd = rays_d / jnp.linalg.norm(rays_d, axis=-1, keepdims=True)
    t = jnp.arange(NI + 1, dtype=jnp.float32) * SS
    tm = 0.5 * (t[:-1] + t[1:])

    px = (rays_o[:, 0:1] + rd[:, 0:1] * tm[None, :]).reshape(P)
    py = (rays_o[:, 1:2] + rd[:, 1:2] * tm[None, :]).reshape(P)
    pz = (rays_o[:, 2:3] + rd[:, 2:3] * tm[None, :]).reshape(P)
    mask = (jnp.sqrt(px * px + py * py + pz * pz) < RAD).reshape(B, NI)
    fpx = (px / RAD + 1.0) * (RES / 2.0)
    fpy = (py / RAD + 1.0) * (RES / 2.0)
    fpz = (pz / RAD + 1.0) * (RES / 2.0)

    def corners(fp):
        lo = jnp.clip(jnp.floor(fp - 0.5), 0.0, RES - 1)
        hi = jnp.clip(jnp.floor(fp + 0.5), 0.0, RES - 1)
        return lo, hi, fp - lo

    pfx0, pfx1, ox = corners(fpx)
    pfy0, pfy1, oy = corners(fpy)
    pfz0, pfz1, oz = corners(fpz)

    rep8 = lambda v: jnp.broadcast_to(v[:, None], (P, 8)).reshape(N1)
    j8 = jnp.arange(N1, dtype=jnp.int32) % 8
    xbit, ybit, zbit = j8 // 4, (j8 // 2) % 2, j8 % 2
    gx = jnp.where(xbit > 0, rep8(pfx1), rep8(pfx0))
    gy = jnp.where(ybit > 0, rep8(pfy1), rep8(pfy0))
    gz = jnp.where(zbit > 0, rep8(pfz1), rep8(pfz0))
    gidxf = (gx.astype(jnp.int32) * (RES * RES)
             + gy.astype(jnp.int32) * RES + gz.astype(jnp.int32))
    fx = jnp.where(xbit > 0, rep8(ox), 1.0 - rep8(ox))
    fy = jnp.where(ybit > 0, rep8(oy), 1.0 - rep8(oy))
    fz = jnp.where(zbit > 0, rep8(oz), 1.0 - rep8(oz))
    w8f = fx * fy * fz

    gvf = _sc_gather_scalar(G.reshape(-1),
                            gidxf.reshape(N1R, T128)).reshape(N1)

    fi = (jnp.clip(gvf, -1.0, 1.0) + 1.0) * (NF / 2.0)
    fl = jnp.clip(jnp.floor(fi), 0.0, NF - 1)
    ce = jnp.clip(jnp.ceil(fi), 0.0, NF - 1)
    wlo = w8f * (ce - fi)
    whi = w8f * (fi - fl)
    rep2 = lambda v: jnp.broadcast_to(v[:, None], (N1, 2)).reshape(N2)
    par = jnp.arange(N2, dtype=jnp.int32) % 2
    wcf = jnp.where(par > 0, rep2(whi), rep2(wlo)).reshape(N2R, T128)
    idx2 = jnp.where(par > 0, rep2(ce), rep2(fl)).astype(
        jnp.int32).reshape(N2R, T128)

    fv = _sc_gather_reduce(F, idx2, wcf)

    enc = _sh16_enc(rd)
    cw1b = jnp.concatenate([jnp.zeros((1, 64), jnp.float32), color_W1[16:]],
                           axis=0)
    maskf = mask.astype(jnp.float32)
    return _tc1(fv, enc, maskf, sigma_W1, sigma_W2, color_W1[:16],
                cw1b, color_W2, color_W3)
